# baseline (device time: 99466 ns/iter reference)
import jax
import jax.numpy as jnp
from jax import lax
from jax.experimental import pallas as pl
from jax.experimental.pallas import tpu as pltpu

N_DEV = 16


def kernel(x, w_mat, scale_x, scale_w):
    m_per, k = x.shape
    _, n_per = w_mat.shape
    half = m_per // 2
    q = m_per // 4

    x8 = x.astype(jnp.float8_e4m3fn)
    w_bf = w_mat.astype(jnp.bfloat16)
    scale = (scale_x * scale_w).astype(jnp.float32)

    def body(x_ref, w_ref, s_ref, out_ref, comm_ref,
             send_ra, send_rb, send_la, send_lb,
             recv_ra, recv_rb, recv_la, recv_lb):
        my = lax.axis_index("i")
        right = lax.rem(my + 1, N_DEV)
        left = lax.rem(my - 1 + N_DEV, N_DEV)
        s = s_ref[0]

        barrier_sem = pltpu.get_barrier_semaphore()
        for nbr in (left, right):
            pl.semaphore_signal(barrier_sem, inc=1, device_id=(nbr,),
                                device_id_type=pl.DeviceIdType.MESH)
        pl.semaphore_wait(barrier_sem, 2)

        def compute(chunk, origin, lo=0):
            acc = lax.dot_general(
                chunk.astype(jnp.bfloat16), w_ref[...],
                (((1,), (0,)), ((), ())),
                preferred_element_type=jnp.float32,
            )
            rows = chunk.shape[0]
            out_ref[pl.ds(origin * m_per + lo, rows), :] = acc * s

        def mk(src, o, lo, qi, dev, ssems, rsems):
            return pltpu.make_async_remote_copy(
                src_ref=src,
                dst_ref=comm_ref.at[o, pl.ds(lo + qi * q, q)],
                send_sem=ssems.at[o, qi],
                recv_sem=rsems.at[o, qi],
                device_id=(dev,),
                device_id_type=pl.DeviceIdType.MESH,
            )

        def fwd(o, lo, qi, dev, ssems, rsems):
            return mk(comm_ref.at[o, pl.ds(lo + qi * q, q)], o, lo, qi,
                      dev, ssems, rsems)

        sends = []

        def start(d):
            d.start()
            sends.append(d)

        for qi in (0, 1):
            start(mk(x_ref.at[pl.ds(qi * q, q)], my, 0, qi,
                     right, send_ra, recv_ra))
            start(mk(x_ref.at[pl.ds(half + qi * q, q)], my, half, qi,
                     left, send_lb, recv_lb))
        for qi in (0, 1):
            start(mk(x_ref.at[pl.ds(half + qi * q, q)], my, half, qi,
                     right, send_rb, recv_rb))
            start(mk(x_ref.at[pl.ds(qi * q, q)], my, 0, qi,
                     left, send_la, recv_la))
        compute(x_ref[...], my)

        for h in range(7):
            o_r = lax.rem(my - 1 - h + N_DEV, N_DEV)
            o_l = lax.rem(my + 1 + h, N_DEV)
            for qi in (0, 1):
                ra = fwd(o_r, 0, qi, right, send_ra, recv_ra)
                ra.wait_recv()
                ra.start()
                sends.append(ra)
                lb = fwd(o_l, half, qi, left, send_lb, recv_lb)
                lb.wait_recv()
                lb.start()
                sends.append(lb)
            for qi in (0, 1):
                rb = fwd(o_r, half, qi, right, send_rb, recv_rb)
                rb.wait_recv()
                if h < 6:
                    rb.start()
                    sends.append(rb)
                la = fwd(o_l, 0, qi, left, send_la, recv_la)
                la.wait_recv()
                if h < 6:
                    la.start()
                    sends.append(la)
            compute(comm_ref[o_r], o_r)
            compute(comm_ref[o_l], o_l)

        o8 = lax.rem(my + 8, N_DEV)
        fwd(o8, 0, 0, right, send_ra, recv_ra).wait_recv()
        fwd(o8, 0, 1, right, send_ra, recv_ra).wait_recv()
        compute(comm_ref[o8, pl.ds(0, half)], o8, 0)
        fwd(o8, half, 0, left, send_lb, recv_lb).wait_recv()
        fwd(o8, half, 1, left, send_lb, recv_lb).wait_recv()
        compute(comm_ref[o8, pl.ds(half, half)], o8, half)

        for d in sends:
            d.wait_send()

    sem2 = pltpu.SemaphoreType.DMA((N_DEV, 2))
    return pl.pallas_call(
        body,
        out_shape=jax.ShapeDtypeStruct((N_DEV * m_per, n_per), jnp.float32),
        in_specs=[
            pl.BlockSpec(memory_space=pltpu.VMEM),
            pl.BlockSpec(memory_space=pltpu.VMEM),
            pl.BlockSpec(memory_space=pltpu.SMEM),
        ],
        out_specs=pl.BlockSpec(memory_space=pltpu.VMEM),
        scratch_shapes=[
            pltpu.VMEM((N_DEV, m_per, k), jnp.float8_e4m3fn),
            sem2, sem2, sem2, sem2,
            sem2, sem2, sem2, sem2,
        ],
        compiler_params=pltpu.CompilerParams(collective_id=0),
    )(x8, w_bf, scale)


# device time: 96469 ns/iter; 1.0311x vs baseline; 1.0311x over previous
import jax
import jax.numpy as jnp
from jax import lax
from jax.experimental import pallas as pl
from jax.experimental.pallas import tpu as pltpu

N_DEV = 16

CYCLE = [0, 4, 8, 12, 13, 9, 5, 1, 2, 6, 10, 14, 15, 11, 7, 3]
CYCLE_POS = [CYCLE.index(i) for i in range(N_DEV)]


def kernel(x, w_mat, scale_x, scale_w):
    m_per, k = x.shape
    _, n_per = w_mat.shape
    half = m_per // 2

    x8 = x.astype(jnp.float8_e4m3fn)
    w_bf = w_mat.astype(jnp.bfloat16)
    scale = (scale_x * scale_w).astype(jnp.float32)
    cyc = jnp.array(CYCLE, dtype=jnp.int32)
    cpos = jnp.array(CYCLE_POS, dtype=jnp.int32)

    def body(x_ref, w_ref, s_ref, cyc_ref, cpos_ref, out_ref, comm_ref,
             send_ra, send_rb, send_la, send_lb,
             recv_ra, recv_rb, recv_la, recv_lb):
        my = lax.axis_index("i")
        pos = cpos_ref[my]
        right = cyc_ref[lax.rem(pos + 1, N_DEV)]
        left = cyc_ref[lax.rem(pos - 1 + N_DEV, N_DEV)]
        s = s_ref[0]

        barrier_sem = pltpu.get_barrier_semaphore()
        for nbr in (left, right):
            pl.semaphore_signal(barrier_sem, inc=1, device_id=(nbr,),
                                device_id_type=pl.DeviceIdType.MESH)
        pl.semaphore_wait(barrier_sem, 2)

        def compute(chunk, origin, lo=0):
            acc = lax.dot_general(
                chunk.astype(jnp.bfloat16), w_ref[...],
                (((1,), (0,)), ((), ())),
                preferred_element_type=jnp.float32,
            )
            rows = chunk.shape[0]
            out_ref[pl.ds(origin * m_per + lo, rows), :] = acc * s

        def mk(src, o, lo, dev, ssems, rsems):
            return pltpu.make_async_remote_copy(
                src_ref=src,
                dst_ref=comm_ref.at[o, pl.ds(lo, half)],
                send_sem=ssems.at[o],
                recv_sem=rsems.at[o],
                device_id=(dev,),
                device_id_type=pl.DeviceIdType.MESH,
            )

        def fwd(o, lo, dev, ssems, rsems):
            return mk(comm_ref.at[o, pl.ds(lo, half)], o, lo, dev,
                      ssems, rsems)

        sends = []

        def start(d):
            d.start()
            sends.append(d)

        xa = x_ref.at[pl.ds(0, half)]
        xb = x_ref.at[pl.ds(half, half)]
        start(mk(xa, my, 0, right, send_ra, recv_ra))
        start(mk(xb, my, half, left, send_lb, recv_lb))
        start(mk(xb, my, half, right, send_rb, recv_rb))
        start(mk(xa, my, 0, left, send_la, recv_la))
        compute(x_ref[...], my)

        for h in range(7):
            o_r = cyc_ref[lax.rem(pos - 1 - h + N_DEV, N_DEV)]
            o_l = cyc_ref[lax.rem(pos + 1 + h, N_DEV)]
            ra = fwd(o_r, 0, right, send_ra, recv_ra)
            ra.wait_recv()
            ra.start()
            sends.append(ra)
            lb = fwd(o_l, half, left, send_lb, recv_lb)
            lb.wait_recv()
            lb.start()
            sends.append(lb)
            rb = fwd(o_r, half, right, send_rb, recv_rb)
            rb.wait_recv()
            if h < 6:
                rb.start()
                sends.append(rb)
            la = fwd(o_l, 0, left, send_la, recv_la)
            la.wait_recv()
            if h < 6:
                la.start()
                sends.append(la)
            compute(comm_ref[o_r], o_r)
            compute(comm_ref[o_l], o_l)

        o8 = cyc_ref[lax.rem(pos + 8, N_DEV)]
        fwd(o8, 0, right, send_ra, recv_ra).wait_recv()
        compute(comm_ref[o8, pl.ds(0, half)], o8, 0)
        fwd(o8, half, left, send_lb, recv_lb).wait_recv()
        compute(comm_ref[o8, pl.ds(half, half)], o8, half)

        for d in sends:
            d.wait_send()

    return pl.pallas_call(
        body,
        out_shape=jax.ShapeDtypeStruct((N_DEV * m_per, n_per), jnp.float32),
        in_specs=[
            pl.BlockSpec(memory_space=pltpu.VMEM),
            pl.BlockSpec(memory_space=pltpu.VMEM),
            pl.BlockSpec(memory_space=pltpu.SMEM),
            pl.BlockSpec(memory_space=pltpu.SMEM),
            pl.BlockSpec(memory_space=pltpu.SMEM),
        ],
        out_specs=pl.BlockSpec(memory_space=pltpu.VMEM),
        scratch_shapes=[
            pltpu.VMEM((N_DEV, m_per, k), jnp.float8_e4m3fn),
            pltpu.SemaphoreType.DMA((N_DEV,)),
            pltpu.SemaphoreType.DMA((N_DEV,)),
            pltpu.SemaphoreType.DMA((N_DEV,)),
            pltpu.SemaphoreType.DMA((N_DEV,)),
            pltpu.SemaphoreType.DMA((N_DEV,)),
            pltpu.SemaphoreType.DMA((N_DEV,)),
            pltpu.SemaphoreType.DMA((N_DEV,)),
            pltpu.SemaphoreType.DMA((N_DEV,)),
        ],
        compiler_params=pltpu.CompilerParams(collective_id=0),
    )(x8, w_bf, scale, cyc, cpos)
